# Initial kernel scaffold; baseline (speedup 1.0000x reference)
import jax
import jax.numpy as jnp
from jax.experimental import pallas as pl


def kernel(x, edge_index, batch, W1, b1, W2, b2):
    raise NotImplementedError("write your pallas kernel here")


# speedup vs baseline: 5.4372x; 5.4372x over previous
import functools

import jax
import jax.numpy as jnp
from jax import lax
from jax.experimental import pallas as pl
from jax.experimental.pallas import tpu as pltpu
from jax.experimental.pallas import tpu_sc as plsc

N_NODES = 10000
N_EDGES = 320000
D = 128
N_GRAPHS = 128

NC, NS = 2, 16
NP = 10240
STRIPE = NP // NS
CW = 128
CHUNK = 128
N_CHUNKS = N_EDGES // CHUNK
CHUNKS_PER_CORE = N_CHUNKS // NC

BM = 1000


def _linear_body(x_ref, w_ref, b_ref, o_ref):
    o_ref[...] = (
        jnp.dot(x_ref[...], w_ref[...], preferred_element_type=jnp.float32)
        + b_ref[...]
    )


def _linear(x, Wt, b):
    return pl.pallas_call(
        _linear_body,
        grid=(N_NODES // BM,),
        in_specs=[
            pl.BlockSpec((BM, D), lambda i: (i, 0)),
            pl.BlockSpec((D, D), lambda i: (0, 0)),
            pl.BlockSpec((1, D), lambda i: (0, 0)),
        ],
        out_specs=pl.BlockSpec((BM, D), lambda i: (i, 0)),
        out_shape=jax.ShapeDtypeStruct((N_NODES, D), jnp.float32),
    )(x, Wt, b.reshape(1, D))


def _mean_relu(p_ref, cnt_ref):
    blk = p_ref[0] + p_ref[1]
    c = cnt_ref[0, :, 0:1] + cnt_ref[1, :, 0:1]
    return jnp.maximum(blk / jnp.maximum(c, 1.0), 0.0)


def _mrl_body(p_ref, cnt_ref, w_ref, b_ref, o_ref):
    z = _mean_relu(p_ref, cnt_ref)
    o_ref[...] = (
        jnp.dot(z, w_ref[...], preferred_element_type=jnp.float32) + b_ref[...]
    )


def _mean_relu_linear(p, cnt, Wt, b):
    return pl.pallas_call(
        _mrl_body,
        grid=(N_NODES // BM,),
        in_specs=[
            pl.BlockSpec((NC, BM, D), lambda i: (0, i, 0)),
            pl.BlockSpec((NC, BM, CW), lambda i: (0, i, 0)),
            pl.BlockSpec((D, D), lambda i: (0, 0)),
            pl.BlockSpec((1, D), lambda i: (0, 0)),
        ],
        out_specs=pl.BlockSpec((BM, D), lambda i: (i, 0)),
        out_shape=jax.ShapeDtypeStruct((N_NODES, D), jnp.float32),
    )(p, cnt, Wt, b.reshape(1, D))


def _readout_body(p_ref, cnt_ref, batch_ref, o_ref, acc_ref, gcnt_ref):
    i = pl.program_id(0)

    @pl.when(i == 0)
    def _():
        acc_ref[...] = jnp.zeros_like(acc_ref)
        gcnt_ref[...] = jnp.zeros_like(gcnt_ref)

    h2 = _mean_relu(p_ref, cnt_ref)
    b = batch_ref[0]
    onehot = (
        lax.broadcasted_iota(jnp.int32, (N_GRAPHS, BM), 0) == b
    ).astype(jnp.float32)
    acc_ref[...] += jnp.dot(onehot, h2, preferred_element_type=jnp.float32)
    gcnt_ref[...] += jnp.dot(
        onehot, jnp.ones((BM, D), jnp.float32), preferred_element_type=jnp.float32
    )

    @pl.when(i == pl.num_programs(0) - 1)
    def _():
        o_ref[...] = acc_ref[...] / jnp.maximum(gcnt_ref[...], 1.0)


def _readout(p, cnt, batch):
    return pl.pallas_call(
        _readout_body,
        grid=(N_NODES // BM,),
        in_specs=[
            pl.BlockSpec((NC, BM, D), lambda i: (0, i, 0)),
            pl.BlockSpec((NC, BM, CW), lambda i: (0, i, 0)),
            pl.BlockSpec((1, 1, BM), lambda i: (i, 0, 0)),
        ],
        out_specs=pl.BlockSpec((N_GRAPHS, D), lambda i: (0, 0)),
        out_shape=jax.ShapeDtypeStruct((N_GRAPHS, D), jnp.float32),
        scratch_shapes=[
            pltpu.VMEM((N_GRAPHS, D), jnp.float32),
            pltpu.VMEM((N_GRAPHS, D), jnp.float32),
        ],
    )(p, cnt, batch.reshape(N_NODES // BM, 1, BM))


_MESH = plsc.VectorSubcoreMesh(
    core_axis_name="c", subcore_axis_name="s", num_cores=NC, num_subcores=NS
)


def _edge_agg(h, src, dst, zrows):
    @functools.partial(
        pl.kernel,
        out_type=jax.ShapeDtypeStruct((NC * NP, D), jnp.float32),
        mesh=_MESH,
        scratch_types=[
            pltpu.VMEM((CHUNK,), jnp.int32),
            pltpu.VMEM((CHUNK,), jnp.int32),
            pltpu.VMEM((CHUNK, D), jnp.float32),
            pltpu.VMEM_SHARED((NP, D), jnp.float32),
            pltpu.SemaphoreType.DMA,
        ],
    )
    def k(h_hbm, src_hbm, dst_hbm, zr_hbm, acc_out,
          sidx, didx, rows, acc_sh, sem):
        cid = lax.axis_index("c")
        sid = lax.axis_index("s")
        r0 = sid * STRIPE
        nblk = STRIPE // CHUNK

        pltpu.sync_copy(zr_hbm, rows)
        for j in range(nblk):
            pltpu.sync_copy(rows, acc_sh.at[pl.ds(r0 + j * CHUNK, CHUNK)])
        plsc.subcore_barrier()

        def body(kk, carry):
            t = sid + NS * kk

            @pl.when(t < CHUNKS_PER_CORE)
            def _():
                off = (cid + NC * t) * CHUNK
                pltpu.sync_copy(src_hbm.at[pl.ds(off, CHUNK)], sidx)
                pltpu.sync_copy(dst_hbm.at[pl.ds(off, CHUNK)], didx)
                pltpu.async_copy(h_hbm.at[sidx], rows, sem).wait()
                pltpu.sync_copy(rows, acc_sh.at[didx], add=True)

            return carry

        nk = (CHUNKS_PER_CORE + NS - 1) // NS
        lax.fori_loop(0, nk, body, 0)

        plsc.subcore_barrier()
        for j in range(nblk):
            o = r0 + j * CHUNK
            pltpu.sync_copy(acc_sh.at[pl.ds(o, CHUNK)], rows)
            pltpu.sync_copy(rows, acc_out.at[pl.ds(cid * NP + o, CHUNK)])

    return k(h, src, dst, zrows).reshape(NC, NP, D)


def _degree_count(dst, zcnt, onesrows):
    @functools.partial(
        pl.kernel,
        out_type=jax.ShapeDtypeStruct((NC * NP, CW), jnp.float32),
        mesh=_MESH,
        scratch_types=[
            pltpu.VMEM((CHUNK,), jnp.int32),
            pltpu.VMEM((CHUNK, CW), jnp.float32),
            pltpu.VMEM((CHUNK, CW), jnp.float32),
            pltpu.VMEM_SHARED((NP, CW), jnp.float32),
            pltpu.SemaphoreType.DMA,
        ],
    )
    def k(dst_hbm, zc_hbm, ones_hbm, cnt_out, didx, onesb, cbuf, cnt_sh, sem):
        cid = lax.axis_index("c")
        sid = lax.axis_index("s")
        r0 = sid * STRIPE
        nblk = STRIPE // CHUNK

        pltpu.sync_copy(zc_hbm, cbuf)
        for j in range(nblk):
            pltpu.sync_copy(cbuf, cnt_sh.at[pl.ds(r0 + j * CHUNK, CHUNK)])
        pltpu.sync_copy(ones_hbm, onesb)
        plsc.subcore_barrier()

        def body(kk, carry):
            t = sid + NS * kk

            @pl.when(t < CHUNKS_PER_CORE)
            def _():
                off = (cid + NC * t) * CHUNK
                pltpu.sync_copy(dst_hbm.at[pl.ds(off, CHUNK)], didx)
                pltpu.sync_copy(onesb, cnt_sh.at[didx], add=True)

            return carry

        nk = (CHUNKS_PER_CORE + NS - 1) // NS
        lax.fori_loop(0, nk, body, 0)

        plsc.subcore_barrier()
        for j in range(nblk):
            o = r0 + j * CHUNK
            pltpu.sync_copy(cnt_sh.at[pl.ds(o, CHUNK)], cbuf)
            pltpu.sync_copy(cbuf, cnt_out.at[pl.ds(cid * NP + o, CHUNK)])

    return k(dst, zcnt, onesrows).reshape(NC, NP, CW)


@jax.jit
def kernel(x, edge_index, batch, W1, b1, W2, b2):
    src = edge_index[0]
    dst = edge_index[1]
    zrows = jnp.zeros((CHUNK, D), jnp.float32)
    zcnt = jnp.zeros((CHUNK, CW), jnp.float32)
    onesrows = jnp.ones((CHUNK, CW), jnp.float32)

    h1 = _linear(x, W1.T, b1)
    cnt = _degree_count(dst, zcnt, onesrows)
    p1 = _edge_agg(h1, src, dst, zrows)
    h2 = _mean_relu_linear(p1, cnt, W2.T, b2)
    p2 = _edge_agg(h2, src, dst, zrows)
    return _readout(p2, cnt, batch)
